# baseline (device time: 56552 ns/iter reference)
import jax
import jax.numpy as jnp
from jax import lax
from jax.experimental import pallas as pl
from jax.experimental.pallas import tpu as pltpu

N_DEV = 8
SQ = 512
D_MODEL = 1024
SKV = 2048
HEADS = 8
DH = 128
SCALE = 0.08838834764831843
HALF = SQ // 2
CHUNK = HALF // N_DEV
SEMS_PER_HALF = 14


def _body(x_ref, wq_ref, wo_ref, k_ref, v_ref, out_ref,
          comm_ref, rs_buf, send_sems, recv_sems):
    my = lax.axis_index("i")

    barrier_sem = pltpu.get_barrier_semaphore()
    for t in range(1, N_DEV):
        pl.semaphore_signal(
            barrier_sem, inc=1,
            device_id=(lax.rem(my + t, N_DEV),),
            device_id_type=pl.DeviceIdType.MESH,
        )
    pl.semaphore_wait(barrier_sem, N_DEV - 1)

    def rs_issue(hh):
        pend = []
        for t in range(1, N_DEV):
            tgt = lax.rem(my + t, N_DEV)
            rdma = pltpu.make_async_remote_copy(
                src_ref=comm_ref.at[hh, tgt],
                dst_ref=rs_buf.at[hh, 7 - t],
                send_sem=send_sems.at[hh * SEMS_PER_HALF + t - 1],
                recv_sem=recv_sems.at[hh * SEMS_PER_HALF + 7 - t],
                device_id=(tgt,),
                device_id_type=pl.DeviceIdType.MESH,
            )
            rdma.start()
            pend.append(rdma)
        return pend

    def rs_finish(hh, pend):
        for rdma in pend:
            rdma.wait()
        red = comm_ref[hh, my].astype(jnp.float32)
        for r in range(N_DEV - 1):
            red = red + rs_buf[hh, r].astype(jnp.float32)
        comm_ref[hh, my] = red.astype(jnp.bfloat16)

    def ag_issue(hh):
        pend = []
        for t in range(1, N_DEV):
            tgt = lax.rem(my + t, N_DEV)
            rdma = pltpu.make_async_remote_copy(
                src_ref=comm_ref.at[hh, my],
                dst_ref=comm_ref.at[hh, my],
                send_sem=send_sems.at[hh * SEMS_PER_HALF + 7 + t - 1],
                recv_sem=recv_sems.at[hh * SEMS_PER_HALF + 7 + 7 - t],
                device_id=(tgt,),
                device_id_type=pl.DeviceIdType.MESH,
            )
            rdma.start()
            pend.append(rdma)
        return pend

    def ag_finish(hh, pend):
        for rdma in pend:
            rdma.wait()
        out_ref[hh] = comm_ref[hh].astype(jnp.float32)

    q_full = jnp.dot(x_ref[...], wq_ref[...],
                     preferred_element_type=jnp.float32)

    def attn_partial(row0, hook=None):
        q = q_full[row0:row0 + HALF, :]
        partial = jnp.zeros((HALF, D_MODEL), jnp.float32)
        for h in range(HEADS):
            sl = slice(h * DH, (h + 1) * DH)
            qh = q[:, sl] * SCALE
            s = lax.dot_general(qh, k_ref[:, sl], (((1,), (1,)), ((), ())),
                                preferred_element_type=jnp.float32)
            p = jnp.exp(s)
            l = jnp.sum(p, axis=-1, keepdims=True)
            o = jnp.dot(p, v_ref[:, sl],
                        preferred_element_type=jnp.float32) / l
            partial = partial + jnp.dot(o, wo_ref[sl, :],
                                        preferred_element_type=jnp.float32)
            if hook is not None:
                hook(h)
        return partial

    def store_half(hh, partial):
        for c in range(N_DEV):
            comm_ref[hh, c] = partial[c * CHUNK:(c + 1) * CHUNK, :].astype(
                jnp.bfloat16)

    partial0 = attn_partial(0)
    store_half(0, partial0)
    state = {"p": rs_issue(0)}

    def hook(h):
        if h == 2:
            rs_finish(0, state["p"])
            state["p"] = ag_issue(0)
        elif h == 5:
            ag_finish(0, state["p"])

    partial1 = attn_partial(HALF, hook)
    store_half(1, partial1)

    pend = rs_issue(1)
    rs_finish(1, pend)
    pend = ag_issue(1)
    ag_finish(1, pend)


def kernel(x, Wq, Wo, K_ext, V_ext):
    x2 = x.reshape(SQ, D_MODEL)
    k2 = K_ext.reshape(SKV, HEADS * DH)
    v2 = V_ext.reshape(SKV, HEADS * DH)

    out = pl.pallas_call(
        _body,
        out_shape=jax.ShapeDtypeStruct((2, N_DEV, CHUNK, D_MODEL),
                                       jnp.float32),
        in_specs=[pl.BlockSpec(memory_space=pltpu.VMEM)] * 5,
        out_specs=pl.BlockSpec(memory_space=pltpu.VMEM),
        scratch_shapes=[
            pltpu.VMEM((2, N_DEV, CHUNK, D_MODEL), jnp.bfloat16),
            pltpu.VMEM((2, N_DEV - 1, CHUNK, D_MODEL), jnp.bfloat16),
            pltpu.SemaphoreType.DMA((2 * SEMS_PER_HALF,)),
            pltpu.SemaphoreType.DMA((2 * SEMS_PER_HALF,)),
        ],
        compiler_params=pltpu.CompilerParams(collective_id=0),
    )(x2, Wq, Wo, k2, v2)
    return out.reshape(1, SQ, D_MODEL)


# device time: 54407 ns/iter; 1.0394x vs baseline; 1.0394x over previous
import jax
import jax.numpy as jnp
from jax import lax
from jax.experimental import pallas as pl
from jax.experimental.pallas import tpu as pltpu

N_DEV = 8
SQ = 512
D_MODEL = 1024
SKV = 2048
HEADS = 8
DH = 128
SCALE = 0.08838834764831843
HALF = SQ // 2
CHUNK = HALF // N_DEV
SEMS_PER_HALF = 14


def _body(x_ref, wq_ref, wo_ref, k_ref, v_ref, out_ref,
          comm_ref, rs_buf, o_scr, send_sems, recv_sems):
    my = lax.axis_index("i")

    barrier_sem = pltpu.get_barrier_semaphore()
    for t in range(1, N_DEV):
        pl.semaphore_signal(
            barrier_sem, inc=1,
            device_id=(lax.rem(my + t, N_DEV),),
            device_id_type=pl.DeviceIdType.MESH,
        )
    pl.semaphore_wait(barrier_sem, N_DEV - 1)

    def rs_issue(hh):
        pend = []
        for t in range(1, N_DEV):
            tgt = lax.rem(my + t, N_DEV)
            rdma = pltpu.make_async_remote_copy(
                src_ref=comm_ref.at[hh, tgt],
                dst_ref=rs_buf.at[hh, 7 - t],
                send_sem=send_sems.at[hh * SEMS_PER_HALF + t - 1],
                recv_sem=recv_sems.at[hh * SEMS_PER_HALF + 7 - t],
                device_id=(tgt,),
                device_id_type=pl.DeviceIdType.MESH,
            )
            rdma.start()
            pend.append(rdma)
        return pend

    def rs_finish(hh, pend):
        for rdma in pend:
            rdma.wait()
        red = comm_ref[hh, my].astype(jnp.float32)
        for r in range(N_DEV - 1):
            red = red + rs_buf[hh, r].astype(jnp.float32)
        comm_ref[hh, my] = red.astype(jnp.bfloat16)

    def ag_issue(hh):
        pend = []
        for t in range(1, N_DEV):
            tgt = lax.rem(my + t, N_DEV)
            rdma = pltpu.make_async_remote_copy(
                src_ref=comm_ref.at[hh, my],
                dst_ref=comm_ref.at[hh, my],
                send_sem=send_sems.at[hh * SEMS_PER_HALF + 7 + t - 1],
                recv_sem=recv_sems.at[hh * SEMS_PER_HALF + 7 + 7 - t],
                device_id=(tgt,),
                device_id_type=pl.DeviceIdType.MESH,
            )
            rdma.start()
            pend.append(rdma)
        return pend

    def ag_finish(hh, pend):
        for rdma in pend:
            rdma.wait()
        out_ref[hh] = comm_ref[hh].astype(jnp.float32)

    q_full = jnp.dot(x_ref[...], wq_ref[...],
                     preferred_element_type=jnp.float32)

    def attn_partial(row0, hook=None):
        q = q_full[row0:row0 + HALF, :]
        for h in range(HEADS):
            sl = slice(h * DH, (h + 1) * DH)
            qh = q[:, sl] * SCALE
            s = lax.dot_general(qh, k_ref[:, sl], (((1,), (1,)), ((), ())),
                                preferred_element_type=jnp.float32)
            p = jnp.exp(s)
            l = jnp.sum(p, axis=-1, keepdims=True)
            o_scr[:, sl] = jnp.dot(p, v_ref[:, sl],
                                   preferred_element_type=jnp.float32) / l
            if hook is not None:
                hook(h)
        return jnp.dot(o_scr[...], wo_ref[...],
                       preferred_element_type=jnp.float32)

    def store_half(hh, partial):
        for c in range(N_DEV):
            comm_ref[hh, c] = partial[c * CHUNK:(c + 1) * CHUNK, :].astype(
                jnp.bfloat16)

    partial0 = attn_partial(0)
    store_half(0, partial0)
    state = {"p": rs_issue(0)}

    def hook(h):
        if h == 2:
            rs_finish(0, state["p"])
            state["p"] = ag_issue(0)
        elif h == 5:
            ag_finish(0, state["p"])

    partial1 = attn_partial(HALF, hook)
    store_half(1, partial1)

    pend = rs_issue(1)
    rs_finish(1, pend)
    pend = ag_issue(1)
    ag_finish(1, pend)


def kernel(x, Wq, Wo, K_ext, V_ext):
    x2 = x.reshape(SQ, D_MODEL)
    k2 = K_ext.reshape(SKV, HEADS * DH)
    v2 = V_ext.reshape(SKV, HEADS * DH)

    out = pl.pallas_call(
        _body,
        out_shape=jax.ShapeDtypeStruct((2, N_DEV, CHUNK, D_MODEL),
                                       jnp.float32),
        in_specs=[pl.BlockSpec(memory_space=pltpu.VMEM)] * 5,
        out_specs=pl.BlockSpec(memory_space=pltpu.VMEM),
        scratch_shapes=[
            pltpu.VMEM((2, N_DEV, CHUNK, D_MODEL), jnp.bfloat16),
            pltpu.VMEM((2, N_DEV - 1, CHUNK, D_MODEL), jnp.bfloat16),
            pltpu.VMEM((HALF, D_MODEL), jnp.float32),
            pltpu.SemaphoreType.DMA((2 * SEMS_PER_HALF,)),
            pltpu.SemaphoreType.DMA((2 * SEMS_PER_HALF,)),
        ],
        compiler_params=pltpu.CompilerParams(collective_id=0),
    )(x2, Wq, Wo, k2, v2)
    return out.reshape(1, SQ, D_MODEL)


# device time: 50309 ns/iter; 1.1241x vs baseline; 1.0815x over previous
import jax
import jax.numpy as jnp
from jax import lax
from jax.experimental import pallas as pl
from jax.experimental.pallas import tpu as pltpu

N_DEV = 8
SQ = 512
D_MODEL = 1024
SKV = 2048
HEADS = 8
DH = 128
SCALE = 0.08838834764831843
HALF = SQ // 2
CHUNK = HALF // N_DEV
SEMS_PER_HALF = 14


def _body(x_ref, wq_ref, wo_ref, k_ref, v_ref, out_ref,
          comm_ref, rs_buf, o_scr, send_sems, recv_sems):
    my = lax.axis_index("i")

    barrier_sem = pltpu.get_barrier_semaphore()
    for t in range(1, N_DEV):
        pl.semaphore_signal(
            barrier_sem, inc=1,
            device_id=(lax.rem(my + t, N_DEV),),
            device_id_type=pl.DeviceIdType.MESH,
        )
    pl.semaphore_wait(barrier_sem, N_DEV - 1)

    def rs_issue(hh):
        pend = []
        for t in range(1, N_DEV):
            tgt = lax.rem(my + t, N_DEV)
            rdma = pltpu.make_async_remote_copy(
                src_ref=comm_ref.at[hh, tgt],
                dst_ref=rs_buf.at[hh, 7 - t],
                send_sem=send_sems.at[hh * SEMS_PER_HALF + t - 1],
                recv_sem=recv_sems.at[hh * SEMS_PER_HALF + 7 - t],
                device_id=(tgt,),
                device_id_type=pl.DeviceIdType.MESH,
            )
            rdma.start()
            pend.append(rdma)
        return pend

    def rs_finish(hh, pend):
        for rdma in pend:
            rdma.wait()
        red = comm_ref[hh, my].astype(jnp.float32)
        for r in range(N_DEV - 1):
            red = red + rs_buf[hh, r].astype(jnp.float32)
        comm_ref[hh, my] = red.astype(jnp.bfloat16)

    def ag_issue(hh):
        pend = []
        for t in range(1, N_DEV):
            tgt = lax.rem(my + t, N_DEV)
            rdma = pltpu.make_async_remote_copy(
                src_ref=comm_ref.at[hh, my],
                dst_ref=comm_ref.at[hh, my],
                send_sem=send_sems.at[hh * SEMS_PER_HALF + 7 + t - 1],
                recv_sem=recv_sems.at[hh * SEMS_PER_HALF + 7 + 7 - t],
                device_id=(tgt,),
                device_id_type=pl.DeviceIdType.MESH,
            )
            rdma.start()
            pend.append(rdma)
        return pend

    def ag_finish(hh, pend):
        out_ref[hh, my] = comm_ref[hh, my].astype(jnp.float32)
        for t, rdma in enumerate(pend, start=1):
            rdma.wait()
            c = lax.rem(my - t + N_DEV, N_DEV)
            out_ref[hh, c] = comm_ref[hh, c].astype(jnp.float32)

    q_full = jnp.dot(x_ref[...], wq_ref[...],
                     preferred_element_type=jnp.float32)

    def attn_partial(row0, hook=None):
        q = q_full[row0:row0 + HALF, :]
        for h in range(HEADS):
            sl = slice(h * DH, (h + 1) * DH)
            qh = q[:, sl] * SCALE
            s = lax.dot_general(qh, k_ref[:, sl], (((1,), (1,)), ((), ())),
                                preferred_element_type=jnp.float32)
            p = jnp.exp(s)
            l = jnp.sum(p, axis=-1, keepdims=True)
            o_scr[:, sl] = jnp.dot(p, v_ref[:, sl],
                                   preferred_element_type=jnp.float32) / l
            if hook is not None:
                hook(h)
        return jnp.dot(o_scr[...], wo_ref[...],
                       preferred_element_type=jnp.float32)

    def store_half(hh, partial):
        for c in range(N_DEV):
            comm_ref[hh, c] = partial[c * CHUNK:(c + 1) * CHUNK, :].astype(
                jnp.bfloat16)

    partial0 = attn_partial(0)
    store_half(0, partial0)
    state = {"p": rs_issue(0)}

    def hook(h):
        if h == 2:
            rs_finish(0, state["p"])
            state["p"] = ag_issue(0)

    partial1 = attn_partial(HALF, hook)
    store_half(1, partial1)

    pend = rs_issue(1)
    ag_finish(0, state["p"])
    rs_finish(1, pend)
    pend = ag_issue(1)
    ag_finish(1, pend)


def kernel(x, Wq, Wo, K_ext, V_ext):
    x2 = x.reshape(SQ, D_MODEL)
    k2 = K_ext.reshape(SKV, HEADS * DH)
    v2 = V_ext.reshape(SKV, HEADS * DH)

    out = pl.pallas_call(
        _body,
        out_shape=jax.ShapeDtypeStruct((2, N_DEV, CHUNK, D_MODEL),
                                       jnp.float32),
        in_specs=[pl.BlockSpec(memory_space=pltpu.VMEM)] * 5,
        out_specs=pl.BlockSpec(memory_space=pltpu.VMEM),
        scratch_shapes=[
            pltpu.VMEM((2, N_DEV, CHUNK, D_MODEL), jnp.bfloat16),
            pltpu.VMEM((2, N_DEV - 1, CHUNK, D_MODEL), jnp.bfloat16),
            pltpu.VMEM((HALF, D_MODEL), jnp.float32),
            pltpu.SemaphoreType.DMA((2 * SEMS_PER_HALF,)),
            pltpu.SemaphoreType.DMA((2 * SEMS_PER_HALF,)),
        ],
        compiler_params=pltpu.CompilerParams(collective_id=0),
    )(x2, Wq, Wo, k2, v2)
    return out.reshape(1, SQ, D_MODEL)


# device time: 48993 ns/iter; 1.1543x vs baseline; 1.0269x over previous
import jax
import jax.numpy as jnp
from jax import lax
from jax.experimental import pallas as pl
from jax.experimental.pallas import tpu as pltpu

N_DEV = 8
SQ = 512
D_MODEL = 1024
SKV = 2048
HEADS = 8
DH = 128
SCALE = 0.08838834764831843
HALF = SQ // 2
CHUNK = HALF // N_DEV
SEMS_PER_HALF = 14


def _body(x_ref, wq_ref, wo_ref, k_ref, v_ref, out_ref,
          comm_ref, rs_buf, o_scr, send_sems, recv_sems):
    my = lax.axis_index("i")

    barrier_sem = pltpu.get_barrier_semaphore()
    for t in range(1, N_DEV):
        pl.semaphore_signal(
            barrier_sem, inc=1,
            device_id=(lax.rem(my + t, N_DEV),),
            device_id_type=pl.DeviceIdType.MESH,
        )

    def rs_issue(hh):
        pend = []
        for t in range(1, N_DEV):
            tgt = lax.rem(my + t, N_DEV)
            rdma = pltpu.make_async_remote_copy(
                src_ref=comm_ref.at[hh, tgt],
                dst_ref=rs_buf.at[hh, 7 - t],
                send_sem=send_sems.at[hh * SEMS_PER_HALF + t - 1],
                recv_sem=recv_sems.at[hh * SEMS_PER_HALF + 7 - t],
                device_id=(tgt,),
                device_id_type=pl.DeviceIdType.MESH,
            )
            rdma.start()
            pend.append(rdma)
        return pend

    def rs_finish(hh, pend):
        for rdma in pend:
            rdma.wait()
        red = comm_ref[hh, my].astype(jnp.float32)
        for r in range(N_DEV - 1):
            red = red + rs_buf[hh, r].astype(jnp.float32)
        comm_ref[hh, my] = red.astype(jnp.bfloat16)

    def ag_issue(hh):
        pend = []
        for t in range(1, N_DEV):
            tgt = lax.rem(my + t, N_DEV)
            rdma = pltpu.make_async_remote_copy(
                src_ref=comm_ref.at[hh, my],
                dst_ref=comm_ref.at[hh, my],
                send_sem=send_sems.at[hh * SEMS_PER_HALF + 7 + t - 1],
                recv_sem=recv_sems.at[hh * SEMS_PER_HALF + 7 + 7 - t],
                device_id=(tgt,),
                device_id_type=pl.DeviceIdType.MESH,
            )
            rdma.start()
            pend.append(rdma)
        return pend

    def ag_finish(hh, pend):
        out_ref[hh, my] = comm_ref[hh, my].astype(jnp.float32)
        for t, rdma in enumerate(pend, start=1):
            rdma.wait()
            c = lax.rem(my - t + N_DEV, N_DEV)
            out_ref[hh, c] = comm_ref[hh, c].astype(jnp.float32)

    q_full = jnp.dot(x_ref[...], wq_ref[...],
                     preferred_element_type=jnp.float32)

    def attn_partial(row0, hook=None):
        q = q_full[row0:row0 + HALF, :]
        for h in range(HEADS):
            sl = slice(h * DH, (h + 1) * DH)
            qh = q[:, sl] * SCALE
            s = lax.dot_general(qh, k_ref[:, sl], (((1,), (1,)), ((), ())),
                                preferred_element_type=jnp.float32)
            p = jnp.exp(s)
            l = jnp.sum(p, axis=-1, keepdims=True)
            o_scr[:, sl] = jnp.dot(p, v_ref[:, sl],
                                   preferred_element_type=jnp.float32) / l
            if hook is not None:
                hook(h)
        return jnp.dot(o_scr[...], wo_ref[...],
                       preferred_element_type=jnp.float32)

    def store_half(hh, partial):
        for c in range(N_DEV):
            comm_ref[hh, c] = partial[c * CHUNK:(c + 1) * CHUNK, :].astype(
                jnp.bfloat16)

    partial0 = attn_partial(0)
    store_half(0, partial0)
    pl.semaphore_wait(barrier_sem, N_DEV - 1)
    state = {"p": rs_issue(0)}

    def hook(h):
        if h == 2:
            rs_finish(0, state["p"])
            state["p"] = ag_issue(0)

    partial1 = attn_partial(HALF, hook)
    store_half(1, partial1)

    pend = rs_issue(1)
    ag_finish(0, state["p"])
    rs_finish(1, pend)
    pend = ag_issue(1)
    ag_finish(1, pend)


def kernel(x, Wq, Wo, K_ext, V_ext):
    x2 = x.reshape(SQ, D_MODEL)
    k2 = K_ext.reshape(SKV, HEADS * DH)
    v2 = V_ext.reshape(SKV, HEADS * DH)

    out = pl.pallas_call(
        _body,
        out_shape=jax.ShapeDtypeStruct((2, N_DEV, CHUNK, D_MODEL),
                                       jnp.float32),
        in_specs=[pl.BlockSpec(memory_space=pltpu.VMEM)] * 5,
        out_specs=pl.BlockSpec(memory_space=pltpu.VMEM),
        scratch_shapes=[
            pltpu.VMEM((2, N_DEV, CHUNK, D_MODEL), jnp.bfloat16),
            pltpu.VMEM((2, N_DEV - 1, CHUNK, D_MODEL), jnp.bfloat16),
            pltpu.VMEM((HALF, D_MODEL), jnp.float32),
            pltpu.SemaphoreType.DMA((2 * SEMS_PER_HALF,)),
            pltpu.SemaphoreType.DMA((2 * SEMS_PER_HALF,)),
        ],
        compiler_params=pltpu.CompilerParams(collective_id=0),
    )(x2, Wq, Wo, k2, v2)
    return out.reshape(1, SQ, D_MODEL)


# device time: 48359 ns/iter; 1.1694x vs baseline; 1.0131x over previous
import jax
import jax.numpy as jnp
from jax import lax
from jax.experimental import pallas as pl
from jax.experimental.pallas import tpu as pltpu

N_DEV = 8
SQ = 512
D_MODEL = 1024
SKV = 2048
HEADS = 8
DH = 128
SCALE = 0.08838834764831843
HALF = SQ // 2
CHUNK = HALF // N_DEV
SEMS_PER_HALF = 14


def _body(x_ref, wq_ref, wo_ref, k_ref, v_ref, out_ref,
          comm_ref, rs_buf, o_scr, send_sems, recv_sems):
    my = lax.axis_index("i")

    barrier_sem = pltpu.get_barrier_semaphore()
    for t in range(1, N_DEV):
        pl.semaphore_signal(
            barrier_sem, inc=1,
            device_id=(lax.rem(my + t, N_DEV),),
            device_id_type=pl.DeviceIdType.MESH,
        )

    def rs_issue(hh):
        pend = []
        for t in range(1, N_DEV):
            tgt = lax.rem(my + t, N_DEV)
            rdma = pltpu.make_async_remote_copy(
                src_ref=comm_ref.at[hh, tgt],
                dst_ref=rs_buf.at[hh, 7 - t],
                send_sem=send_sems.at[hh * SEMS_PER_HALF + t - 1],
                recv_sem=recv_sems.at[hh * SEMS_PER_HALF + 7 - t],
                device_id=(tgt,),
                device_id_type=pl.DeviceIdType.MESH,
            )
            rdma.start()
            pend.append(rdma)
        return pend

    def rs_finish(hh, pend):
        for rdma in pend:
            rdma.wait()
        red = comm_ref[hh, my].astype(jnp.float32)
        for r in range(N_DEV - 1):
            red = red + rs_buf[hh, r].astype(jnp.float32)
        comm_ref[hh, my] = red.astype(jnp.bfloat16)

    def ag_issue(hh):
        pend = []
        for t in range(1, N_DEV):
            tgt = lax.rem(my + t, N_DEV)
            rdma = pltpu.make_async_remote_copy(
                src_ref=comm_ref.at[hh, my],
                dst_ref=comm_ref.at[hh, my],
                send_sem=send_sems.at[hh * SEMS_PER_HALF + 7 + t - 1],
                recv_sem=recv_sems.at[hh * SEMS_PER_HALF + 7 + 7 - t],
                device_id=(tgt,),
                device_id_type=pl.DeviceIdType.MESH,
            )
            rdma.start()
            pend.append(rdma)
        return pend

    def ag_finish(hh, pend):
        out_ref[hh, my] = comm_ref[hh, my].astype(jnp.float32)
        for t, rdma in enumerate(pend, start=1):
            rdma.wait()
            c = lax.rem(my - t + N_DEV, N_DEV)
            out_ref[hh, c] = comm_ref[hh, c].astype(jnp.float32)

    q_full = jnp.dot(x_ref[...], wq_ref[...],
                     preferred_element_type=jnp.float32)

    def attn_partial(row0, hook=None):
        q = q_full[row0:row0 + HALF, :]
        for h in range(HEADS):
            sl = slice(h * DH, (h + 1) * DH)
            qh = q[:, sl] * SCALE
            s = lax.dot_general(qh, k_ref[:, sl], (((1,), (1,)), ((), ())),
                                preferred_element_type=jnp.float32)
            p = jnp.exp(s)
            l = jnp.sum(p, axis=-1, keepdims=True)
            o_scr[:, sl] = jnp.dot(p, v_ref[:, sl],
                                   preferred_element_type=jnp.float32) / l
            if hook is not None:
                hook(h)
        return jnp.dot(o_scr[...], wo_ref[...],
                       preferred_element_type=jnp.float32)

    def store_half(hh, partial):
        for c in range(N_DEV):
            comm_ref[hh, c] = partial[c * CHUNK:(c + 1) * CHUNK, :].astype(
                jnp.bfloat16)

    partial0 = attn_partial(0)
    store_half(0, partial0)
    pl.semaphore_wait(barrier_sem, N_DEV - 1)
    state = {"p": rs_issue(0)}

    def hook(h):
        if h == 3:
            rs_finish(0, state["p"])
            state["p"] = ag_issue(0)

    partial1 = attn_partial(HALF, hook)
    store_half(1, partial1)

    pend = rs_issue(1)
    ag_finish(0, state["p"])
    rs_finish(1, pend)
    pend = ag_issue(1)
    ag_finish(1, pend)


def kernel(x, Wq, Wo, K_ext, V_ext):
    x2 = x.reshape(SQ, D_MODEL)
    k2 = K_ext.reshape(SKV, HEADS * DH)
    v2 = V_ext.reshape(SKV, HEADS * DH)

    out = pl.pallas_call(
        _body,
        out_shape=jax.ShapeDtypeStruct((2, N_DEV, CHUNK, D_MODEL),
                                       jnp.float32),
        in_specs=[pl.BlockSpec(memory_space=pltpu.VMEM)] * 5,
        out_specs=pl.BlockSpec(memory_space=pltpu.VMEM),
        scratch_shapes=[
            pltpu.VMEM((2, N_DEV, CHUNK, D_MODEL), jnp.bfloat16),
            pltpu.VMEM((2, N_DEV - 1, CHUNK, D_MODEL), jnp.bfloat16),
            pltpu.VMEM((HALF, D_MODEL), jnp.float32),
            pltpu.SemaphoreType.DMA((2 * SEMS_PER_HALF,)),
            pltpu.SemaphoreType.DMA((2 * SEMS_PER_HALF,)),
        ],
        compiler_params=pltpu.CompilerParams(collective_id=0),
    )(x2, Wq, Wo, k2, v2)
    return out.reshape(1, SQ, D_MODEL)
